# baseline (speedup 1.0000x reference)
import functools
import math

import jax
import jax.numpy as jnp
from jax import lax
from jax.experimental import pallas as pl
from jax.experimental.pallas import tpu as pltpu
from jax.experimental.pallas import tpu_sc as plsc

_N_LEVELS = 16
_F = 2
_HASHMAP = 2 ** 19
_BASE = 16
_MAXR = 512
_PLS = math.exp(math.log(_MAXR / _BASE) / (_N_LEVELS - 1))
_RES = []
_OFF = [0]
_tot = 0
for _i in range(_N_LEVELS):
    _r = math.floor(_BASE * _PLS ** _i)
    _RES.append(_r)
    _tot += min(_HASHMAP, (_r + 1) ** 3)
    _OFF.append(_tot)
_N_TOTAL = _tot
_P1 = 2654435761
_P2 = 805459861

_B = 262144
_NW = 32
_PPW = _B // _NW
_CHUNK = 128
_NCHUNK = _PPW // _CHUNK
_NGRP = _CHUNK // 16
_OUTC = 3 + 2 * _N_LEVELS
_NPAD = -(-_N_TOTAL // 4) * 4
_MBLK = _NPAD // 4


def _body(xh, yh, zh, e_hbm, out_hbm,
          x_v, y_v, z_v,
          idxA, idxB, subA, subB,
          rowsA, rowsB,
          fxA, fyA, fzA, fxB, fyB, fzB,
          outb_v, semA, semB):
    idx_bufs = (idxA, idxB)
    sub_bufs = (subA, subB)
    row_bufs = (rowsA, rowsB)
    frac_bufs = ((fxA, fyA, fzA), (fxB, fyB, fzB))
    sems = (semA, semB)
    coord_refs = (x_v, y_v, z_v)
    nc = 2
    wid = lax.axis_index("s") * nc + lax.axis_index("c")
    base = wid * _PPW
    pltpu.sync_copy(xh.at[pl.ds(base, _PPW)], x_v)
    pltpu.sync_copy(yh.at[pl.ds(base, _PPW)], y_v)
    pltpu.sync_copy(zh.at[pl.ds(base, _PPW)], z_v)
    iota = lax.iota(jnp.int32, 16)

    def idx_pass(cb, lvl):
        bsel = lvl % 2
        idx_v = idx_bufs[bsel]
        sub_v = sub_bufs[bsel]
        frac_refs = frac_bufs[bsel]
        resf = jnp.float32(_RES[lvl])
        resm1 = jnp.int32(_RES[lvl] - 1)
        size = _OFF[lvl + 1] - _OFF[lvl]
        off0 = _OFF[lvl]

        def body(g, c2):
            s = cb + g * 16
            hparts = []
            for d, prime in ((0, 1), (1, _P1), (2, _P2)):
                u = coord_refs[d][pl.ds(s, 16)]
                p = u * resf
                vi = jnp.minimum(p.astype(jnp.int32), resm1)
                fr = p - vi.astype(jnp.float32)
                frac_refs[d][pl.ds(g * 16, 16)] = fr
                uv = vi.astype(jnp.uint32)
                h0 = uv * jnp.uint32(prime)
                h1 = h0 + jnp.uint32(prime)
                hparts.append((h0, h1))
            for c in range(8):
                hx = hparts[0][(c >> 2) & 1]
                hy = hparts[1][(c >> 1) & 1]
                hz = hparts[2][c & 1]
                h = hx ^ hy ^ hz
                if size == _HASHMAP:
                    hm = h & jnp.uint32(size - 1)
                else:
                    hm = h % jnp.uint32(size)
                grow = hm.astype(jnp.int32) + jnp.int32(off0)
                idx_v[pl.ds(c * _CHUNK + g * 16, 16)] = lax.shift_right_logical(
                    grow, 2)
                sub_v[c, pl.ds(g * 16, 16)] = (grow & jnp.int32(3)) * 2
            return c2

        lax.fori_loop(0, _NGRP, body, 0)

    def fire(lvl):
        bsel = lvl % 2
        return [
            pltpu.async_copy(
                e_hbm.at[idx_bufs[bsel]], row_bufs[bsel], sems[bsel]),
        ]

    def mac_pass(lvl):
        bsel = lvl % 2
        rows_v = row_bufs[bsel]
        sub_v = sub_bufs[bsel]
        fx_v, fy_v, fz_v = frac_bufs[bsel]

        def body(g, c2):
            ridx = g * 16 + iota
            fx = fx_v[pl.ds(g * 16, 16)]
            fy = fy_v[pl.ds(g * 16, 16)]
            fz = fz_v[pl.ds(g * 16, 16)]
            gx = 1.0 - fx
            gy = 1.0 - fy
            gz = 1.0 - fz
            wxy = (gx * gy, gx * fy, fx * gy, fx * fy)
            acc0 = jnp.zeros((16,), jnp.float32)
            acc1 = jnp.zeros((16,), jnp.float32)
            for c in range(8):
                wc = wxy[c >> 1] * (fz if (c & 1) else gz)
                col = sub_v[c, pl.ds(g * 16, 16)]
                rrow = ridx + (c * _CHUNK)
                e0 = plsc.load_gather(rows_v, [rrow, col])
                e1 = plsc.load_gather(rows_v, [rrow, col + 1])
                acc0 = acc0 + wc * e0
                acc1 = acc1 + wc * e1
            rbase = ridx * _OUTC
            plsc.store_scatter(outb_v, [rbase + (3 + 2 * lvl)], acc0)
            plsc.store_scatter(outb_v, [rbase + (4 + 2 * lvl)], acc1)
            return c2

        lax.fori_loop(0, _NGRP, body, 0)

    def chunk_body(ch, carry):
        cb = ch * _CHUNK

        def xyz_store(g, c2):
            rbase = (g * 16 + iota) * _OUTC
            for d in range(3):
                v = coord_refs[d][pl.ds(cb + g * 16, 16)]
                plsc.store_scatter(outb_v, [rbase + d], v)
            return c2

        lax.fori_loop(0, _NGRP, xyz_store, 0)

        idx_pass(cb, 0)
        cps = fire(0)
        for lvl in range(1, _N_LEVELS):
            idx_pass(cb, lvl)
            cps_next = fire(lvl)
            for cp in cps:
                cp.wait()
            mac_pass(lvl - 1)
            cps = cps_next
        for cp in cps:
            cp.wait()
        mac_pass(_N_LEVELS - 1)

        pltpu.sync_copy(
            outb_v,
            out_hbm.at[pl.ds((base + cb) * _OUTC, _CHUNK * _OUTC)])
        return carry

    lax.fori_loop(0, _NCHUNK, chunk_body, 0)


_mesh = plsc.VectorSubcoreMesh(core_axis_name="c", subcore_axis_name="s")

_grid_kernel = functools.partial(
    pl.kernel,
    mesh=_mesh,
    compiler_params=pltpu.CompilerParams(
        needs_layout_passes=False, use_tc_tiling_on_sc=False),
    out_type=jax.ShapeDtypeStruct((_B * _OUTC,), jnp.float32),
    scratch_types=(
        [pltpu.VMEM((_PPW,), jnp.float32)] * 3
        + [pltpu.VMEM((8 * _CHUNK,), jnp.int32)] * 2
        + [pltpu.VMEM((8, _CHUNK), jnp.int32)] * 2
        + [pltpu.VMEM((8 * _CHUNK, 8), jnp.float32)] * 2
        + [pltpu.VMEM((_CHUNK,), jnp.float32)] * 6
        + [pltpu.VMEM((_CHUNK * _OUTC,), jnp.float32),
           pltpu.SemaphoreType.DMA, pltpu.SemaphoreType.DMA]
    ),
)(_body)


def kernel(xyz, embeddings):
    x = xyz[:, 0]
    y = xyz[:, 1]
    z = xyz[:, 2]
    pad = _NPAD - _N_TOTAL
    e = jnp.pad(embeddings, ((0, pad), (0, 0))).reshape(_MBLK, 8)
    flat = _grid_kernel(x, y, z, e)
    return flat.reshape(_B, _OUTC)


# speedup vs baseline: 1.9512x; 1.9512x over previous
import functools
import math

import jax
import jax.numpy as jnp
from jax import lax
from jax.experimental import pallas as pl
from jax.experimental.pallas import tpu as pltpu
from jax.experimental.pallas import tpu_sc as plsc

_N_LEVELS = 16
_F = 2
_HASHMAP = 2 ** 19
_BASE = 16
_MAXR = 512
_PLS = math.exp(math.log(_MAXR / _BASE) / (_N_LEVELS - 1))
_RES = []
_OFF = [0]
_tot = 0
for _i in range(_N_LEVELS):
    _r = math.floor(_BASE * _PLS ** _i)
    _RES.append(_r)
    _tot += min(_HASHMAP, (_r + 1) ** 3)
    _OFF.append(_tot)
_N_TOTAL = _tot
_P1 = 2654435761
_P2 = 805459861

_B = 262144
_NW = 32
_PPW = _B // _NW
_CHUNK = 128
_NCHUNK = _PPW // _CHUNK
_NGRP = _CHUNK // 16
_OUTC = 3 + 2 * _N_LEVELS
_NPAD = -(-_N_TOTAL // 4) * 4
_MBLK = _NPAD // 4


def _body(xh, yh, zh, e_hbm, out_hbm,
          x_v, y_v, z_v,
          idxA, idxB, subA, subB,
          rowsA, rowsB,
          fxA, fyA, fzA, fxB, fyB, fzB,
          outb_v, semA, semB):
    idx_bufs = (idxA, idxB)
    sub_bufs = (subA, subB)
    row_bufs = (rowsA, rowsB)
    frac_bufs = ((fxA, fyA, fzA), (fxB, fyB, fzB))
    sems = (semA, semB)
    coord_refs = (x_v, y_v, z_v)
    nc = 2
    wid = lax.axis_index("s") * nc + lax.axis_index("c")
    base = wid * _PPW
    pltpu.sync_copy(xh.at[pl.ds(base, _PPW)], x_v)
    pltpu.sync_copy(yh.at[pl.ds(base, _PPW)], y_v)
    pltpu.sync_copy(zh.at[pl.ds(base, _PPW)], z_v)
    iota = lax.iota(jnp.int32, 16)

    def idx_pass(cb, lvl):
        bsel = lvl % 2
        idx_v = idx_bufs[bsel]
        sub_v = sub_bufs[bsel]
        frac_refs = frac_bufs[bsel]
        resf = jnp.float32(_RES[lvl])
        resm1 = jnp.int32(_RES[lvl] - 1)
        size = _OFF[lvl + 1] - _OFF[lvl]
        off0 = _OFF[lvl]

        def body(g, c2):
            s = cb + g * 16
            hparts = []
            for d, prime in ((0, 1), (1, _P1), (2, _P2)):
                u = coord_refs[d][pl.ds(s, 16)]
                p = u * resf
                vi = jnp.minimum(p.astype(jnp.int32), resm1)
                fr = p - vi.astype(jnp.float32)
                frac_refs[d][pl.ds(g * 16, 16)] = fr
                uv = vi.astype(jnp.uint32)
                h0 = uv * jnp.uint32(prime)
                h1 = h0 + jnp.uint32(prime)
                hparts.append((h0, h1))
            for c in range(8):
                hx = hparts[0][(c >> 2) & 1]
                hy = hparts[1][(c >> 1) & 1]
                hz = hparts[2][c & 1]
                h = hx ^ hy ^ hz
                if size == _HASHMAP:
                    hm = h & jnp.uint32(size - 1)
                else:
                    hm = h % jnp.uint32(size)
                grow = hm.astype(jnp.int32) + jnp.int32(off0)
                idx_v[pl.ds(c * _CHUNK + g * 16, 16)] = lax.shift_right_logical(
                    grow, 2)
                sub_v[c, pl.ds(g * 16, 16)] = grow & jnp.int32(3)
            return c2

        lax.fori_loop(0, _NGRP, body, 0)

    def fire(lvl):
        bsel = lvl % 2
        return [
            pltpu.async_copy(
                e_hbm.at[idx_bufs[bsel]], row_bufs[bsel], sems[bsel]),
        ]

    def mac_pass(lvl):
        bsel = lvl % 2
        rows_v = row_bufs[bsel]
        sub_v = sub_bufs[bsel]
        fx_v, fy_v, fz_v = frac_bufs[bsel]

        def body(g, c2):
            ridx = g * 16 + iota
            fx = fx_v[pl.ds(g * 16, 16)]
            fy = fy_v[pl.ds(g * 16, 16)]
            fz = fz_v[pl.ds(g * 16, 16)]
            gx = 1.0 - fx
            gy = 1.0 - fy
            gz = 1.0 - fz
            wxy = (gx * gy, gx * fy, fx * gy, fx * fy)
            acc0 = jnp.zeros((16,), jnp.float32)
            acc1 = jnp.zeros((16,), jnp.float32)
            for c in range(8):
                wc = wxy[c >> 1] * (fz if (c & 1) else gz)
                col = sub_v[c, pl.ds(g * 16, 16)]
                rrow = ridx + (c * _CHUNK)
                e0 = plsc.load_gather(rows_v, [rrow, col])
                e1 = plsc.load_gather(rows_v, [rrow, col + 4])
                acc0 = acc0 + wc * e0
                acc1 = acc1 + wc * e1
            rbase = ridx * _OUTC
            plsc.store_scatter(outb_v, [rbase + (3 + 2 * lvl)], acc0)
            plsc.store_scatter(outb_v, [rbase + (4 + 2 * lvl)], acc1)
            return c2

        lax.fori_loop(0, _NGRP, body, 0)

    def chunk_body(ch, carry):
        cb = ch * _CHUNK

        def xyz_store(g, c2):
            rbase = (g * 16 + iota) * _OUTC
            for d in range(3):
                v = coord_refs[d][pl.ds(cb + g * 16, 16)]
                plsc.store_scatter(outb_v, [rbase + d], v)
            return c2

        lax.fori_loop(0, _NGRP, xyz_store, 0)

        idx_pass(cb, 0)
        cps = fire(0)
        for lvl in range(1, _N_LEVELS):
            idx_pass(cb, lvl)
            cps_next = fire(lvl)
            for cp in cps:
                cp.wait()
            mac_pass(lvl - 1)
            cps = cps_next
        for cp in cps:
            cp.wait()
        mac_pass(_N_LEVELS - 1)

        pltpu.sync_copy(
            outb_v,
            out_hbm.at[pl.ds((base + cb) * _OUTC, _CHUNK * _OUTC)])
        return carry

    lax.fori_loop(0, _NCHUNK, chunk_body, 0)


_mesh = plsc.VectorSubcoreMesh(core_axis_name="c", subcore_axis_name="s")

_grid_kernel = functools.partial(
    pl.kernel,
    mesh=_mesh,
    compiler_params=pltpu.CompilerParams(
        needs_layout_passes=False, use_tc_tiling_on_sc=False),
    out_type=jax.ShapeDtypeStruct((_B * _OUTC,), jnp.float32),
    scratch_types=(
        [pltpu.VMEM((_PPW,), jnp.float32)] * 3
        + [pltpu.VMEM((8 * _CHUNK,), jnp.int32)] * 2
        + [pltpu.VMEM((8, _CHUNK), jnp.int32)] * 2
        + [pltpu.VMEM((8 * _CHUNK, 8), jnp.float32)] * 2
        + [pltpu.VMEM((_CHUNK,), jnp.float32)] * 6
        + [pltpu.VMEM((_CHUNK * _OUTC,), jnp.float32),
           pltpu.SemaphoreType.DMA, pltpu.SemaphoreType.DMA]
    ),
)(_body)


def kernel(xyz, embeddings):
    x = xyz[:, 0]
    y = xyz[:, 1]
    z = xyz[:, 2]
    pad = _NPAD - _N_TOTAL
    e0 = jnp.pad(embeddings[:, 0], (0, pad)).reshape(_MBLK, 4)
    e1 = jnp.pad(embeddings[:, 1], (0, pad)).reshape(_MBLK, 4)
    e = jnp.concatenate([e0, e1], axis=1)
    flat = _grid_kernel(x, y, z, e)
    return flat.reshape(_B, _OUTC)


# speedup vs baseline: 4.1774x; 2.1410x over previous
import functools
import math

import jax
import jax.numpy as jnp
from jax import lax
from jax.experimental import pallas as pl
from jax.experimental.pallas import tpu as pltpu
from jax.experimental.pallas import tpu_sc as plsc

_N_LEVELS = 16
_F = 2
_HASHMAP = 2 ** 19
_BASE = 16
_MAXR = 512
_PLS = math.exp(math.log(_MAXR / _BASE) / (_N_LEVELS - 1))
_RES = []
_OFF = [0]
_tot = 0
for _i in range(_N_LEVELS):
    _r = math.floor(_BASE * _PLS ** _i)
    _RES.append(_r)
    _tot += min(_HASHMAP, (_r + 1) ** 3)
    _OFF.append(_tot)
_N_TOTAL = _tot
_P1 = 2654435761
_P2 = 805459861

_B = 262144
_NW = 32
_PPW = _B // _NW
_CHUNK = 128
_NCHUNK = _PPW // _CHUNK
_NGRP = _CHUNK // 16
_OUTC = 3 + 2 * _N_LEVELS
_ICHUNK = 4096
_NPAD = -(-_N_TOTAL // (_NW * _ICHUNK)) * (_NW * _ICHUNK)
_MBLK = _NPAD // 4
_IPW = _NPAD // _NW
_ISTEPS = _IPW // _ICHUNK
_IGRP = _ICHUNK // 16


def _body(xh, yh, zh, e_hbm, out_hbm,
          x_v, y_v, z_v,
          idxA, idxB, subA, subB,
          rowsA, rowsB,
          fxA, fyA, fzA, fxB, fyB, fzB,
          outb_v, semA, semB):
    idx_bufs = (idxA, idxB)
    sub_bufs = (subA, subB)
    row_bufs = (rowsA, rowsB)
    frac_bufs = ((fxA, fyA, fzA), (fxB, fyB, fzB))
    sems = (semA, semB)
    coord_refs = (x_v, y_v, z_v)
    nc = 2
    wid = lax.axis_index("s") * nc + lax.axis_index("c")
    base = wid * _PPW
    pltpu.sync_copy(xh.at[pl.ds(base, _PPW)], x_v)
    pltpu.sync_copy(yh.at[pl.ds(base, _PPW)], y_v)
    pltpu.sync_copy(zh.at[pl.ds(base, _PPW)], z_v)
    iota = lax.iota(jnp.int32, 16)

    def idx_pass(cb, lvl):
        bsel = lvl % 2
        idx_v = idx_bufs[bsel]
        sub_v = sub_bufs[bsel]
        frac_refs = frac_bufs[bsel]
        resf = jnp.float32(_RES[lvl])
        resm1 = jnp.int32(_RES[lvl] - 1)
        size = _OFF[lvl + 1] - _OFF[lvl]
        off0 = _OFF[lvl]

        def body(g, c2):
            s = cb + g * 16
            hparts = []
            for d, prime in ((0, 1), (1, _P1), (2, _P2)):
                u = coord_refs[d][pl.ds(s, 16)]
                p = u * resf
                vi = jnp.minimum(p.astype(jnp.int32), resm1)
                fr = p - vi.astype(jnp.float32)
                frac_refs[d][pl.ds(g * 16, 16)] = fr
                uv = vi.astype(jnp.uint32)
                h0 = uv * jnp.uint32(prime)
                h1 = h0 + jnp.uint32(prime)
                hparts.append((h0, h1))
            for c in range(8):
                hx = hparts[0][(c >> 2) & 1]
                hy = hparts[1][(c >> 1) & 1]
                hz = hparts[2][c & 1]
                h = hx ^ hy ^ hz
                if size == _HASHMAP:
                    hm = h & jnp.uint32(size - 1)
                else:
                    hm = h % jnp.uint32(size)
                grow = hm.astype(jnp.int32) + jnp.int32(off0)
                idx_v[pl.ds(c * _CHUNK + g * 16, 16)] = lax.shift_right_logical(
                    grow, 2)
                sub_v[c, pl.ds(g * 16, 16)] = grow & jnp.int32(3)
            return c2

        lax.fori_loop(0, _NGRP, body, 0)

    def fire(lvl):
        bsel = lvl % 2
        return [
            pltpu.async_copy(
                e_hbm.at[idx_bufs[bsel]], row_bufs[bsel], sems[bsel]),
        ]

    def mac_pass(lvl):
        bsel = lvl % 2
        rows_v = row_bufs[bsel]
        sub_v = sub_bufs[bsel]
        fx_v, fy_v, fz_v = frac_bufs[bsel]

        def body(g, c2):
            ridx = g * 16 + iota
            fx = fx_v[pl.ds(g * 16, 16)]
            fy = fy_v[pl.ds(g * 16, 16)]
            fz = fz_v[pl.ds(g * 16, 16)]
            gx = 1.0 - fx
            gy = 1.0 - fy
            gz = 1.0 - fz
            wxy = (gx * gy, gx * fy, fx * gy, fx * fy)
            acc0 = jnp.zeros((16,), jnp.float32)
            acc1 = jnp.zeros((16,), jnp.float32)
            for c in range(8):
                wc = wxy[c >> 1] * (fz if (c & 1) else gz)
                col = sub_v[c, pl.ds(g * 16, 16)]
                rrow = ridx + (c * _CHUNK)
                e0 = plsc.load_gather(rows_v, [rrow, col])
                e1 = plsc.load_gather(rows_v, [rrow, col + 4])
                acc0 = acc0 + wc * e0
                acc1 = acc1 + wc * e1
            rbase = ridx * _OUTC
            plsc.store_scatter(outb_v, [rbase + (3 + 2 * lvl)], acc0)
            plsc.store_scatter(outb_v, [rbase + (4 + 2 * lvl)], acc1)
            return c2

        lax.fori_loop(0, _NGRP, body, 0)

    def chunk_body(ch, carry):
        cb = ch * _CHUNK

        def xyz_store(g, c2):
            rbase = (g * 16 + iota) * _OUTC
            for d in range(3):
                v = coord_refs[d][pl.ds(cb + g * 16, 16)]
                plsc.store_scatter(outb_v, [rbase + d], v)
            return c2

        lax.fori_loop(0, _NGRP, xyz_store, 0)

        idx_pass(cb, 0)
        cps = fire(0)
        for lvl in range(1, _N_LEVELS):
            idx_pass(cb, lvl)
            cps_next = fire(lvl)
            for cp in cps:
                cp.wait()
            mac_pass(lvl - 1)
            cps = cps_next
        for cp in cps:
            cp.wait()
        mac_pass(_N_LEVELS - 1)

        pltpu.sync_copy(
            outb_v,
            out_hbm.at[pl.ds((base + cb) * _OUTC, _CHUNK * _OUTC)])
        return carry

    lax.fori_loop(0, _NCHUNK, chunk_body, 0)


def _inter_body(e0h, e1h, out_hbm,
                a0, a1, b0, b1, obA, obB, semA, semB, semO):
    in_bufs = ((a0, a1), (b0, b1))
    out_bufs = (obA, obB)
    sems = (semA, semB)
    nc = 2
    wid = lax.axis_index("s") * nc + lax.axis_index("c")
    base = wid * _IPW
    iota = lax.iota(jnp.int32, 16)
    dbase = lax.shift_right_logical(iota, 2) * 8 + (iota & jnp.int32(3))

    def load(step, bsel):
        off = base + step * _ICHUNK
        return [
            pltpu.async_copy(e0h.at[pl.ds(off, _ICHUNK)],
                             in_bufs[bsel][0], sems[bsel]),
            pltpu.async_copy(e1h.at[pl.ds(off, _ICHUNK)],
                             in_bufs[bsel][1], sems[bsel]),
        ]

    cps = load(0, 0)
    ocp = None
    for step in range(_ISTEPS):
        bsel = step % 2
        if step + 1 < _ISTEPS:
            cps_next = load(step + 1, 1 - bsel)
        else:
            cps_next = []
        for cp in cps:
            cp.wait()
        v0r, v1r = in_bufs[bsel]
        ob = out_bufs[bsel]

        def body(g, c2):
            dst = dbase + g * 32
            plsc.store_scatter(ob, [dst], v0r[pl.ds(g * 16, 16)])
            plsc.store_scatter(ob, [dst + 4], v1r[pl.ds(g * 16, 16)])
            return c2

        lax.fori_loop(0, _IGRP, body, 0)
        if ocp is not None:
            ocp.wait()
        ocp = pltpu.async_copy(
            ob, out_hbm.at[pl.ds((base + step * _ICHUNK) * 2, 2 * _ICHUNK)],
            semO)
        cps = cps_next
    ocp.wait()


_mesh = plsc.VectorSubcoreMesh(core_axis_name="c", subcore_axis_name="s")

_inter_kernel = functools.partial(
    pl.kernel,
    mesh=_mesh,
    compiler_params=pltpu.CompilerParams(
        needs_layout_passes=False, use_tc_tiling_on_sc=False),
    out_type=jax.ShapeDtypeStruct((_NPAD * 2,), jnp.float32),
    scratch_types=(
        [pltpu.VMEM((_ICHUNK,), jnp.float32)] * 4
        + [pltpu.VMEM((2 * _ICHUNK,), jnp.float32)] * 2
        + [pltpu.SemaphoreType.DMA] * 3
    ),
)(_inter_body)

_grid_kernel = functools.partial(
    pl.kernel,
    mesh=_mesh,
    compiler_params=pltpu.CompilerParams(
        needs_layout_passes=False, use_tc_tiling_on_sc=False),
    out_type=jax.ShapeDtypeStruct((_B * _OUTC,), jnp.float32),
    scratch_types=(
        [pltpu.VMEM((_PPW,), jnp.float32)] * 3
        + [pltpu.VMEM((8 * _CHUNK,), jnp.int32)] * 2
        + [pltpu.VMEM((8, _CHUNK), jnp.int32)] * 2
        + [pltpu.VMEM((8 * _CHUNK, 8), jnp.float32)] * 2
        + [pltpu.VMEM((_CHUNK,), jnp.float32)] * 6
        + [pltpu.VMEM((_CHUNK * _OUTC,), jnp.float32),
           pltpu.SemaphoreType.DMA, pltpu.SemaphoreType.DMA]
    ),
)(_body)


def kernel(xyz, embeddings):
    x = xyz[:, 0]
    y = xyz[:, 1]
    z = xyz[:, 2]
    pad = _NPAD - _N_TOTAL
    e0 = jnp.pad(embeddings[:, 0], (0, pad))
    e1 = jnp.pad(embeddings[:, 1], (0, pad))
    e = _inter_kernel(e0, e1).reshape(_MBLK, 8)
    flat = _grid_kernel(x, y, z, e)
    return flat.reshape(_B, _OUTC)


# speedup vs baseline: 4.4130x; 1.0564x over previous
import functools
import math

import jax
import jax.numpy as jnp
from jax import lax
from jax.experimental import pallas as pl
from jax.experimental.pallas import tpu as pltpu
from jax.experimental.pallas import tpu_sc as plsc

_N_LEVELS = 16
_F = 2
_HASHMAP = 2 ** 19
_BASE = 16
_MAXR = 512
_PLS = math.exp(math.log(_MAXR / _BASE) / (_N_LEVELS - 1))
_RES = []
_OFF = [0]
_tot = 0
for _i in range(_N_LEVELS):
    _r = math.floor(_BASE * _PLS ** _i)
    _RES.append(_r)
    _tot += min(_HASHMAP, (_r + 1) ** 3)
    _OFF.append(_tot)
_N_TOTAL = _tot
_P1 = 2654435761
_P2 = 805459861

_B = 262144
_NW = 32
_PPW = _B // _NW
_CHUNK = 128
_NCHUNK = _PPW // _CHUNK
_NGRP = _CHUNK // 16
_OUTC = 3 + 2 * _N_LEVELS
_ICHUNK = 4096
_NPAD = -(-_N_TOTAL // (_NW * _ICHUNK)) * (_NW * _ICHUNK)
_MBLK = _NPAD // 4
_IPW = _NPAD // _NW
_ISTEPS = _IPW // _ICHUNK
_IGRP = _ICHUNK // 16


def _body(xh, yh, zh, e_hbm, out_hbm,
          x_v, y_v, z_v,
          idxA, idxB, idxC, subA, subB, subC,
          rowsA, rowsB, rowsC,
          fxA, fyA, fzA, fxB, fyB, fzB, fxC, fyC, fzC,
          outb_v, semA, semB, semC):
    idx_bufs = (idxA, idxB, idxC)
    sub_bufs = (subA, subB, subC)
    row_bufs = (rowsA, rowsB, rowsC)
    frac_bufs = ((fxA, fyA, fzA), (fxB, fyB, fzB), (fxC, fyC, fzC))
    sems = (semA, semB, semC)
    coord_refs = (x_v, y_v, z_v)
    nc = 2
    wid = lax.axis_index("s") * nc + lax.axis_index("c")
    base = wid * _PPW
    pltpu.sync_copy(xh.at[pl.ds(base, _PPW)], x_v)
    pltpu.sync_copy(yh.at[pl.ds(base, _PPW)], y_v)
    pltpu.sync_copy(zh.at[pl.ds(base, _PPW)], z_v)
    iota = lax.iota(jnp.int32, 16)

    def idx_pass(cb, lvl):
        bsel = lvl % 3
        idx_v = idx_bufs[bsel]
        sub_v = sub_bufs[bsel]
        frac_refs = frac_bufs[bsel]
        resf = jnp.float32(_RES[lvl])
        resm1 = jnp.int32(_RES[lvl] - 1)
        size = _OFF[lvl + 1] - _OFF[lvl]
        off0 = _OFF[lvl]

        def body(g, c2):
            s = cb + g * 16
            hparts = []
            for d, prime in ((0, 1), (1, _P1), (2, _P2)):
                u = coord_refs[d][pl.ds(s, 16)]
                p = u * resf
                vi = jnp.minimum(p.astype(jnp.int32), resm1)
                fr = p - vi.astype(jnp.float32)
                frac_refs[d][pl.ds(g * 16, 16)] = fr
                uv = vi.astype(jnp.uint32)
                h0 = uv * jnp.uint32(prime)
                h1 = h0 + jnp.uint32(prime)
                hparts.append((h0, h1))
            for c in range(8):
                hx = hparts[0][(c >> 2) & 1]
                hy = hparts[1][(c >> 1) & 1]
                hz = hparts[2][c & 1]
                h = hx ^ hy ^ hz
                if size == _HASHMAP:
                    hm = h & jnp.uint32(size - 1)
                else:
                    hm = h % jnp.uint32(size)
                grow = hm.astype(jnp.int32) + jnp.int32(off0)
                idx_v[pl.ds(c * _CHUNK + g * 16, 16)] = lax.shift_right_logical(
                    grow, 2)
                sub_v[c, pl.ds(g * 16, 16)] = grow & jnp.int32(3)
            return c2

        lax.fori_loop(0, _NGRP, body, 0)

    def fire(lvl):
        bsel = lvl % 3
        return [
            pltpu.async_copy(
                e_hbm.at[idx_bufs[bsel]], row_bufs[bsel], sems[bsel]),
        ]

    def mac_pass(lvl):
        bsel = lvl % 3
        rows_v = row_bufs[bsel]
        sub_v = sub_bufs[bsel]
        fx_v, fy_v, fz_v = frac_bufs[bsel]

        def body(g, c2):
            ridx = g * 16 + iota
            fx = fx_v[pl.ds(g * 16, 16)]
            fy = fy_v[pl.ds(g * 16, 16)]
            fz = fz_v[pl.ds(g * 16, 16)]
            gx = 1.0 - fx
            gy = 1.0 - fy
            gz = 1.0 - fz
            wxy = (gx * gy, gx * fy, fx * gy, fx * fy)
            acc0 = jnp.zeros((16,), jnp.float32)
            acc1 = jnp.zeros((16,), jnp.float32)
            for c in range(8):
                wc = wxy[c >> 1] * (fz if (c & 1) else gz)
                col = sub_v[c, pl.ds(g * 16, 16)]
                rrow = ridx + (c * _CHUNK)
                e0 = plsc.load_gather(rows_v, [rrow, col])
                e1 = plsc.load_gather(rows_v, [rrow, col + 4])
                acc0 = acc0 + wc * e0
                acc1 = acc1 + wc * e1
            rbase = ridx * _OUTC
            plsc.store_scatter(outb_v, [rbase + (3 + 2 * lvl)], acc0)
            plsc.store_scatter(outb_v, [rbase + (4 + 2 * lvl)], acc1)
            return c2

        lax.fori_loop(0, _NGRP, body, 0)

    def chunk_body(ch, carry):
        cb = ch * _CHUNK

        def xyz_store(g, c2):
            rbase = (g * 16 + iota) * _OUTC
            for d in range(3):
                v = coord_refs[d][pl.ds(cb + g * 16, 16)]
                plsc.store_scatter(outb_v, [rbase + d], v)
            return c2

        lax.fori_loop(0, _NGRP, xyz_store, 0)

        inflight = []
        for lvl in range(_N_LEVELS):
            idx_pass(cb, lvl)
            inflight.append(fire(lvl))
            if lvl >= 2:
                for cp in inflight.pop(0):
                    cp.wait()
                mac_pass(lvl - 2)
        for back in (2, 1):
            for cp in inflight.pop(0):
                cp.wait()
            mac_pass(_N_LEVELS - back)

        pltpu.sync_copy(
            outb_v,
            out_hbm.at[pl.ds((base + cb) * _OUTC, _CHUNK * _OUTC)])
        return carry

    lax.fori_loop(0, _NCHUNK, chunk_body, 0)


def _inter_body(e0h, e1h, out_hbm,
                a0, a1, b0, b1, obA, obB, semA, semB, semO):
    in_bufs = ((a0, a1), (b0, b1))
    out_bufs = (obA, obB)
    sems = (semA, semB)
    nc = 2
    wid = lax.axis_index("s") * nc + lax.axis_index("c")
    base = wid * _IPW
    iota = lax.iota(jnp.int32, 16)
    dbase = lax.shift_right_logical(iota, 2) * 8 + (iota & jnp.int32(3))

    def load(step, bsel):
        off = base + step * _ICHUNK
        return [
            pltpu.async_copy(e0h.at[pl.ds(off, _ICHUNK)],
                             in_bufs[bsel][0], sems[bsel]),
            pltpu.async_copy(e1h.at[pl.ds(off, _ICHUNK)],
                             in_bufs[bsel][1], sems[bsel]),
        ]

    cps = load(0, 0)
    ocp = None
    for step in range(_ISTEPS):
        bsel = step % 2
        if step + 1 < _ISTEPS:
            cps_next = load(step + 1, 1 - bsel)
        else:
            cps_next = []
        for cp in cps:
            cp.wait()
        v0r, v1r = in_bufs[bsel]
        ob = out_bufs[bsel]

        def body(g, c2):
            dst = dbase + g * 32
            plsc.store_scatter(ob, [dst], v0r[pl.ds(g * 16, 16)])
            plsc.store_scatter(ob, [dst + 4], v1r[pl.ds(g * 16, 16)])
            return c2

        lax.fori_loop(0, _IGRP, body, 0)
        if ocp is not None:
            ocp.wait()
        ocp = pltpu.async_copy(
            ob, out_hbm.at[pl.ds((base + step * _ICHUNK) * 2, 2 * _ICHUNK)],
            semO)
        cps = cps_next
    ocp.wait()


_mesh = plsc.VectorSubcoreMesh(core_axis_name="c", subcore_axis_name="s")

_inter_kernel = functools.partial(
    pl.kernel,
    mesh=_mesh,
    compiler_params=pltpu.CompilerParams(
        needs_layout_passes=False, use_tc_tiling_on_sc=False),
    out_type=jax.ShapeDtypeStruct((_NPAD * 2,), jnp.float32),
    scratch_types=(
        [pltpu.VMEM((_ICHUNK,), jnp.float32)] * 4
        + [pltpu.VMEM((2 * _ICHUNK,), jnp.float32)] * 2
        + [pltpu.SemaphoreType.DMA] * 3
    ),
)(_inter_body)

_grid_kernel = functools.partial(
    pl.kernel,
    mesh=_mesh,
    compiler_params=pltpu.CompilerParams(
        needs_layout_passes=False, use_tc_tiling_on_sc=False),
    out_type=jax.ShapeDtypeStruct((_B * _OUTC,), jnp.float32),
    scratch_types=(
        [pltpu.VMEM((_PPW,), jnp.float32)] * 3
        + [pltpu.VMEM((8 * _CHUNK,), jnp.int32)] * 3
        + [pltpu.VMEM((8, _CHUNK), jnp.int32)] * 3
        + [pltpu.VMEM((8 * _CHUNK, 8), jnp.float32)] * 3
        + [pltpu.VMEM((_CHUNK,), jnp.float32)] * 9
        + [pltpu.VMEM((_CHUNK * _OUTC,), jnp.float32)]
        + [pltpu.SemaphoreType.DMA] * 3
    ),
)(_body)


def kernel(xyz, embeddings):
    x = xyz[:, 0]
    y = xyz[:, 1]
    z = xyz[:, 2]
    pad = _NPAD - _N_TOTAL
    e0 = jnp.pad(embeddings[:, 0], (0, pad))
    e1 = jnp.pad(embeddings[:, 1], (0, pad))
    e = _inter_kernel(e0, e1).reshape(_MBLK, 8)
    flat = _grid_kernel(x, y, z, e)
    return flat.reshape(_B, _OUTC)
